# Initial kernel scaffold; baseline (speedup 1.0000x reference)
import jax
import jax.numpy as jnp
from jax.experimental import pallas as pl


def kernel(node_attr, edge_attr, edge_index, edge_contact_attr, edge_contact_index, W1, b1, W2, b2):
    raise NotImplementedError("write your pallas kernel here")


# speedup vs baseline: 3.7032x; 3.7032x over previous
import functools

import jax
import jax.numpy as jnp
from jax import lax
from jax.experimental import pallas as pl
from jax.experimental.pallas import tpu as pltpu
from jax.experimental.pallas import tpu_sc as plsc

NC = 2
NS = 16
NW = NC * NS
CHUNK = 80
ZROWS = 128


def _sc_segment_sums(edge_attr, rcv, cont_attr, crcv, num_nodes):
  d = edge_attr.shape[1]
  e = edge_attr.shape[0]
  ec = cont_attr.shape[0]
  nch_e = e // CHUNK
  nch_c = ec // CHUNK
  per_w_e = -(-nch_e // NW)
  per_w_c = -(-nch_c // NW)
  rows_per_tile = -(-num_nodes // (NS * ZROWS)) * ZROWS
  num_nodes = rows_per_tile * NS

  zeros_big = jnp.zeros((ZROWS, d), jnp.float32)
  zeros_cnt = jnp.zeros((ZROWS, 16), jnp.float32)
  ones_cnt = jnp.ones((CHUNK, 16), jnp.float32)

  mesh = plsc.VectorSubcoreMesh(core_axis_name="c", subcore_axis_name="s")

  @functools.partial(
      pl.kernel,
      out_type=(
          jax.ShapeDtypeStruct((NC, num_nodes, d), jnp.float32),
          jax.ShapeDtypeStruct((NC, num_nodes, 16), jnp.float32),
          jax.ShapeDtypeStruct((NC, num_nodes, d), jnp.float32),
          jax.ShapeDtypeStruct((NC, num_nodes, 16), jnp.float32),
      ),
      mesh=mesh,
      compiler_params=pltpu.CompilerParams(use_tc_tiling_on_sc=False),
      scratch_types=[
          pltpu.VMEM_SHARED((num_nodes, d), jnp.float32),
          pltpu.VMEM_SHARED((num_nodes, 16), jnp.float32),
          pltpu.VMEM((CHUNK,), jnp.int32),
          pltpu.VMEM((CHUNK, d), jnp.float32),
          pltpu.VMEM((CHUNK, 16), jnp.float32),
          pltpu.VMEM((ZROWS, d), jnp.float32),
          pltpu.VMEM((ZROWS, 16), jnp.float32),
      ],
  )
  def seg_kernel(eattr, ercv, cattr, crcv_h, zb_hbm, zc_hbm, ones_hbm,
                 msum, mcnt, csum, ccnt,
                 acc, cnt, idx_v, rows_v, ones_v, stage, cstage):
    c = lax.axis_index("c")
    s = lax.axis_index("s")
    w = c * NS + s
    row0 = s * rows_per_tile
    nz = rows_per_tile // ZROWS

    pltpu.sync_copy(ones_hbm, ones_v)

    def zero_acc():
      pltpu.sync_copy(zb_hbm, stage)
      pltpu.sync_copy(zc_hbm, cstage)
      for k in range(nz):
        pltpu.sync_copy(stage, acc.at[pl.ds(row0 + k * ZROWS, ZROWS), :])
        pltpu.sync_copy(cstage, cnt.at[pl.ds(row0 + k * ZROWS, ZROWS), :])

    def scatter_phase(attr_hbm, rcv_hbm, per_w, nch):
      def body(j, carry):
        cid = j * NW + w

        @pl.when(cid < nch)
        def _():
          base = cid * CHUNK
          pltpu.sync_copy(rcv_hbm.at[pl.ds(base, CHUNK)], idx_v)
          pltpu.sync_copy(attr_hbm.at[pl.ds(base, CHUNK), :], rows_v)
          pltpu.sync_copy(rows_v, acc.at[idx_v], add=True)
          pltpu.sync_copy(ones_v, cnt.at[idx_v], add=True)

        return carry

      lax.fori_loop(0, per_w, body, 0)

    def dump(sum_out, cnt_out):
      for k in range(nz):
        r = row0 + k * ZROWS
        pltpu.sync_copy(acc.at[pl.ds(r, ZROWS), :], stage)
        pltpu.sync_copy(stage, sum_out.at[c, pl.ds(r, ZROWS), :])
        pltpu.sync_copy(cnt.at[pl.ds(r, ZROWS), :], cstage)
        pltpu.sync_copy(cstage, cnt_out.at[c, pl.ds(r, ZROWS), :])

    zero_acc()
    plsc.subcore_barrier()
    scatter_phase(eattr, ercv, per_w_e, nch_e)
    plsc.subcore_barrier()
    dump(msum, mcnt)
    zero_acc()
    plsc.subcore_barrier()
    scatter_phase(cattr, crcv_h, per_w_c, nch_c)
    plsc.subcore_barrier()
    dump(csum, ccnt)

  return seg_kernel(edge_attr, rcv, cont_attr, crcv,
                    zeros_big, zeros_cnt, ones_cnt)


def _mlp(node_attr, msum, mcnt, csum, ccnt, W1, b1, W2, b2, block_n=2000):
  n, d = node_attr.shape

  def mlp_kernel(x_ref, ms_ref, mc_ref, cs_ref, cc_ref,
                 w1_ref, b1_ref, w2_ref, b2_ref, o_ref):
    ms = ms_ref[0] + ms_ref[1]
    mc = mc_ref[0, :, 0:1] + mc_ref[1, :, 0:1]
    cs = cs_ref[0] + cs_ref[1]
    cc = cc_ref[0, :, 0:1] + cc_ref[1, :, 0:1]
    aggm = ms / jnp.maximum(mc, 1.0)
    aggc = cs / jnp.maximum(cc, 1.0)
    x = x_ref[...]
    w1 = w1_ref[...]
    h = (jnp.dot(x, w1[0:d], preferred_element_type=jnp.float32)
         + jnp.dot(aggm, w1[d:2 * d], preferred_element_type=jnp.float32)
         + jnp.dot(aggc, w1[2 * d:3 * d], preferred_element_type=jnp.float32)
         + b1_ref[...])
    h = jnp.maximum(h, 0.0)
    o_ref[...] = (jnp.dot(h, w2_ref[...], preferred_element_type=jnp.float32)
                  + b2_ref[...])

  return pl.pallas_call(
      mlp_kernel,
      grid=(n // block_n,),
      in_specs=[
          pl.BlockSpec((block_n, d), lambda i: (i, 0)),
          pl.BlockSpec((NC, block_n, d), lambda i: (0, i, 0)),
          pl.BlockSpec((NC, block_n, 16), lambda i: (0, i, 0)),
          pl.BlockSpec((NC, block_n, d), lambda i: (0, i, 0)),
          pl.BlockSpec((NC, block_n, 16), lambda i: (0, i, 0)),
          pl.BlockSpec((3 * d, d), lambda i: (0, 0)),
          pl.BlockSpec((1, d), lambda i: (0, 0)),
          pl.BlockSpec((d, d), lambda i: (0, 0)),
          pl.BlockSpec((1, d), lambda i: (0, 0)),
      ],
      out_specs=pl.BlockSpec((block_n, d), lambda i: (i, 0)),
      out_shape=jax.ShapeDtypeStruct((n, d), jnp.float32),
  )(node_attr, msum, mcnt, csum, ccnt,
    W1, b1.reshape(1, d), W2, b2.reshape(1, d))


def kernel(node_attr, edge_attr, edge_index, edge_contact_attr,
           edge_contact_index, W1, b1, W2, b2):
  num_nodes = node_attr.shape[0]
  rcv = edge_index[1]
  crcv = edge_contact_index[1]
  msum, mcnt, csum, ccnt = _sc_segment_sums(
      edge_attr, rcv, edge_contact_attr, crcv, num_nodes)
  return _mlp(node_attr, msum, mcnt, csum, ccnt, W1, b1, W2, b2)


# speedup vs baseline: 5.6639x; 1.5295x over previous
import functools

import jax
import jax.numpy as jnp
from jax import lax
from jax.experimental import pallas as pl
from jax.experimental.pallas import tpu as pltpu
from jax.experimental.pallas import tpu_sc as plsc

NC = 2
NS = 16
NW = NC * NS
CHUNK = 80
ZROWS = 80


def _sc_segment_sums(edge_attr, rcv, cont_attr, crcv, num_nodes):
  d = edge_attr.shape[1]
  e = edge_attr.shape[0]
  ec = cont_attr.shape[0]
  nch_e = e // CHUNK
  nch_c = ec // CHUNK
  per_w_e = -(-nch_e // NW)
  per_w_c = -(-nch_c // NW)
  rows_per_tile = -(-num_nodes // (NS * ZROWS)) * ZROWS
  num_nodes = rows_per_tile * NS

  zeros_big = jnp.zeros((ZROWS, d), jnp.float32)
  zeros_cnt = jnp.zeros((ZROWS, 16), jnp.float32)
  ones_cnt = jnp.ones((CHUNK, 16), jnp.float32)

  mesh = plsc.VectorSubcoreMesh(core_axis_name="c", subcore_axis_name="s")

  @functools.partial(
      pl.kernel,
      out_type=(
          jax.ShapeDtypeStruct((NC, num_nodes, d), jnp.float32),
          jax.ShapeDtypeStruct((NC, num_nodes, 16), jnp.float32),
          jax.ShapeDtypeStruct((NC, num_nodes, d), jnp.float32),
          jax.ShapeDtypeStruct((NC, num_nodes, 16), jnp.float32),
      ),
      mesh=mesh,
      compiler_params=pltpu.CompilerParams(use_tc_tiling_on_sc=False),
      scratch_types=[
          pltpu.VMEM_SHARED((num_nodes, d), jnp.float32),
          pltpu.VMEM_SHARED((num_nodes, 16), jnp.float32),
          pltpu.VMEM((2, CHUNK), jnp.int32),
          pltpu.VMEM((2, CHUNK, d), jnp.float32),
          pltpu.VMEM((CHUNK, 16), jnp.float32),
          pltpu.VMEM((ZROWS, d), jnp.float32),
          pltpu.VMEM((ZROWS, 16), jnp.float32),
          pltpu.SemaphoreType.DMA((2,)),
          pltpu.SemaphoreType.DMA((2,)),
      ],
  )
  def seg_kernel(eattr, ercv, cattr, crcv_h, zb_hbm, zc_hbm, ones_hbm,
                 msum, mcnt, csum, ccnt,
                 acc, cnt, idx_v, rows_v, ones_v, stage, cstage, isem, rsem):
    c = lax.axis_index("c")
    s = lax.axis_index("s")
    w = c * NS + s
    row0 = s * rows_per_tile
    nz = rows_per_tile // ZROWS

    pltpu.sync_copy(ones_hbm, ones_v)

    def zero_acc():
      pltpu.sync_copy(zb_hbm, stage)
      pltpu.sync_copy(zc_hbm, cstage)
      for k in range(nz):
        pltpu.sync_copy(stage, acc.at[pl.ds(row0 + k * ZROWS, ZROWS), :])
        pltpu.sync_copy(cstage, cnt.at[pl.ds(row0 + k * ZROWS, ZROWS), :])

    def scatter_phase(attr_hbm, rcv_hbm, per_w, nch):
      def issue(j, b):
        cid = j * NW + w

        @pl.when(cid < nch)
        def _():
          base = cid * CHUNK
          pltpu.async_copy(rcv_hbm.at[pl.ds(base, CHUNK)], idx_v.at[b],
                           isem.at[b])
          pltpu.async_copy(attr_hbm.at[pl.ds(base, CHUNK), :], rows_v.at[b],
                           rsem.at[b])

      def wait(j, b):
        cid = j * NW + w

        @pl.when(cid < nch)
        def _():
          base = cid * CHUNK
          pltpu.make_async_copy(rcv_hbm.at[pl.ds(base, CHUNK)], idx_v.at[b],
                                isem.at[b]).wait()
          pltpu.make_async_copy(attr_hbm.at[pl.ds(base, CHUNK), :],
                                rows_v.at[b], rsem.at[b]).wait()

      def scat(j, b):
        cid = j * NW + w

        @pl.when(cid < nch)
        def _():
          pltpu.sync_copy(rows_v.at[b], acc.at[idx_v.at[b]], add=True)
          pltpu.sync_copy(ones_v, cnt.at[idx_v.at[b]], add=True)

      issue(0, 0)

      def body(g, carry):
        j0 = g * 2
        wait(j0, 0)
        issue(j0 + 1, 1)
        scat(j0, 0)
        wait(j0 + 1, 1)
        issue(j0 + 2, 0)
        scat(j0 + 1, 1)
        return carry

      lax.fori_loop(0, -(-per_w // 2), body, 0)

    def dump(sum_out, cnt_out):
      for k in range(nz):
        r = row0 + k * ZROWS
        pltpu.sync_copy(acc.at[pl.ds(r, ZROWS), :], stage)
        pltpu.sync_copy(stage, sum_out.at[c, pl.ds(r, ZROWS), :])
        pltpu.sync_copy(cnt.at[pl.ds(r, ZROWS), :], cstage)
        pltpu.sync_copy(cstage, cnt_out.at[c, pl.ds(r, ZROWS), :])

    zero_acc()
    plsc.subcore_barrier()
    scatter_phase(eattr, ercv, per_w_e, nch_e)
    plsc.subcore_barrier()
    dump(msum, mcnt)
    zero_acc()
    plsc.subcore_barrier()
    scatter_phase(cattr, crcv_h, per_w_c, nch_c)
    plsc.subcore_barrier()
    dump(csum, ccnt)

  return seg_kernel(edge_attr, rcv, cont_attr, crcv,
                    zeros_big, zeros_cnt, ones_cnt)


def _mlp(node_attr, msum, mcnt, csum, ccnt, W1, b1, W2, b2, block_n=2000):
  n, d = node_attr.shape

  def mlp_kernel(x_ref, ms_ref, mc_ref, cs_ref, cc_ref,
                 w1_ref, b1_ref, w2_ref, b2_ref, o_ref):
    ms = ms_ref[0] + ms_ref[1]
    mc = mc_ref[0, :, 0:1] + mc_ref[1, :, 0:1]
    cs = cs_ref[0] + cs_ref[1]
    cc = cc_ref[0, :, 0:1] + cc_ref[1, :, 0:1]
    aggm = ms / jnp.maximum(mc, 1.0)
    aggc = cs / jnp.maximum(cc, 1.0)
    x = x_ref[...]
    w1 = w1_ref[...]
    h = (jnp.dot(x, w1[0:d], preferred_element_type=jnp.float32)
         + jnp.dot(aggm, w1[d:2 * d], preferred_element_type=jnp.float32)
         + jnp.dot(aggc, w1[2 * d:3 * d], preferred_element_type=jnp.float32)
         + b1_ref[...])
    h = jnp.maximum(h, 0.0)
    o_ref[...] = (jnp.dot(h, w2_ref[...], preferred_element_type=jnp.float32)
                  + b2_ref[...])

  return pl.pallas_call(
      mlp_kernel,
      grid=(n // block_n,),
      in_specs=[
          pl.BlockSpec((block_n, d), lambda i: (i, 0)),
          pl.BlockSpec((NC, block_n, d), lambda i: (0, i, 0)),
          pl.BlockSpec((NC, block_n, 16), lambda i: (0, i, 0)),
          pl.BlockSpec((NC, block_n, d), lambda i: (0, i, 0)),
          pl.BlockSpec((NC, block_n, 16), lambda i: (0, i, 0)),
          pl.BlockSpec((3 * d, d), lambda i: (0, 0)),
          pl.BlockSpec((1, d), lambda i: (0, 0)),
          pl.BlockSpec((d, d), lambda i: (0, 0)),
          pl.BlockSpec((1, d), lambda i: (0, 0)),
      ],
      out_specs=pl.BlockSpec((block_n, d), lambda i: (i, 0)),
      out_shape=jax.ShapeDtypeStruct((n, d), jnp.float32),
  )(node_attr, msum, mcnt, csum, ccnt,
    W1, b1.reshape(1, d), W2, b2.reshape(1, d))


def kernel(node_attr, edge_attr, edge_index, edge_contact_attr,
           edge_contact_index, W1, b1, W2, b2):
  num_nodes = node_attr.shape[0]
  rcv = edge_index[1]
  crcv = edge_contact_index[1]
  msum, mcnt, csum, ccnt = _sc_segment_sums(
      edge_attr, rcv, edge_contact_attr, crcv, num_nodes)
  return _mlp(node_attr, msum, mcnt, csum, ccnt, W1, b1, W2, b2)


# speedup vs baseline: 6.6646x; 1.1767x over previous
import functools

import jax
import jax.numpy as jnp
from jax import lax
from jax.experimental import pallas as pl
from jax.experimental.pallas import tpu as pltpu
from jax.experimental.pallas import tpu_sc as plsc

NC = 2
NS = 16
NW = NC * NS
CHUNK = 128
ZROWS = 128


def _sc_segment_sums(edge_attr, rcv, cont_attr, crcv, num_nodes):
  d = edge_attr.shape[1]
  e = edge_attr.shape[0]
  ec = cont_attr.shape[0]
  nch_e = e // CHUNK
  nch_c = ec // CHUNK
  per_w_e = -(-nch_e // NW)
  per_w_c = -(-nch_c // NW)
  rows_per_tile = -(-num_nodes // (NS * ZROWS)) * ZROWS
  num_nodes = rows_per_tile * NS

  zeros_big = jnp.zeros((ZROWS, d), jnp.float32)
  zeros_cnt = jnp.zeros((ZROWS, 16), jnp.float32)
  ones_cnt = jnp.ones((CHUNK, 16), jnp.float32)

  mesh = plsc.VectorSubcoreMesh(core_axis_name="c", subcore_axis_name="s")

  @functools.partial(
      pl.kernel,
      out_type=(
          jax.ShapeDtypeStruct((NC, num_nodes, d), jnp.float32),
          jax.ShapeDtypeStruct((NC, num_nodes, 16), jnp.float32),
          jax.ShapeDtypeStruct((NC, num_nodes, d), jnp.float32),
          jax.ShapeDtypeStruct((NC, num_nodes, 16), jnp.float32),
      ),
      mesh=mesh,
      compiler_params=pltpu.CompilerParams(use_tc_tiling_on_sc=False),
      scratch_types=[
          pltpu.VMEM_SHARED((num_nodes, d), jnp.float32),
          pltpu.VMEM_SHARED((num_nodes, 16), jnp.float32),
          pltpu.VMEM((2, CHUNK), jnp.int32),
          pltpu.VMEM((2, CHUNK, d), jnp.float32),
          pltpu.VMEM((CHUNK, 16), jnp.float32),
          pltpu.VMEM((ZROWS, 16), jnp.float32),
          pltpu.SemaphoreType.DMA((2,)),
          pltpu.SemaphoreType.DMA((2,)),
          pltpu.SemaphoreType.DMA((2,)),
          pltpu.SemaphoreType.DMA((2,)),
          pltpu.SemaphoreType.DMA((2,)),
      ],
  )
  def seg_kernel(eattr, ercv, cattr, crcv_h, zb_hbm, zc_hbm, ones_hbm,
                 msum, mcnt, csum, ccnt,
                 acc, cnt, idx_v, rows_v, ones_v, cstage,
                 isem, rsem, ssem, csem, wsem):
    c = lax.axis_index("c")
    s = lax.axis_index("s")
    w = c * NS + s
    row0 = s * rows_per_tile
    nz = rows_per_tile // ZROWS

    pltpu.sync_copy(ones_hbm, ones_v)

    def zero_acc():
      pltpu.sync_copy(zb_hbm, rows_v.at[0])
      pltpu.sync_copy(zc_hbm, cstage)
      descs = []
      for k in range(nz):
        r = pl.ds(row0 + k * ZROWS, ZROWS)
        descs.append(pltpu.async_copy(rows_v.at[0], acc.at[r, :],
                                      wsem.at[0]))
        descs.append(pltpu.async_copy(cstage, cnt.at[r, :], wsem.at[1]))
      for dsc in descs:
        dsc.wait()

    def scatter_phase(attr_hbm, rcv_hbm, per_w, nch):
      def issue_load(j, b):
        cid = j * NW + w

        @pl.when(cid < nch)
        def _():
          base = cid * CHUNK
          pltpu.async_copy(rcv_hbm.at[pl.ds(base, CHUNK)], idx_v.at[b],
                           isem.at[b])
          pltpu.async_copy(attr_hbm.at[pl.ds(base, CHUNK), :], rows_v.at[b],
                           rsem.at[b])

      def wait_load(j, b):
        cid = j * NW + w

        @pl.when(cid < nch)
        def _():
          base = cid * CHUNK
          pltpu.make_async_copy(rcv_hbm.at[pl.ds(base, CHUNK)], idx_v.at[b],
                                isem.at[b]).wait()
          pltpu.make_async_copy(attr_hbm.at[pl.ds(base, CHUNK), :],
                                rows_v.at[b], rsem.at[b]).wait()

      def start_scat(j, b):
        cid = j * NW + w

        @pl.when(cid < nch)
        def _():
          pltpu.async_copy(rows_v.at[b], acc.at[idx_v.at[b]], ssem.at[b],
                           add=True)
          pltpu.async_copy(ones_v, cnt.at[idx_v.at[b]], csem.at[b],
                           add=True)

      def wait_scat(j, b):
        cid = j * NW + w

        @pl.when(cid < nch)
        def _():
          pltpu.make_async_copy(rows_v.at[b], acc.at[idx_v.at[b]],
                                ssem.at[b]).wait()
          pltpu.make_async_copy(ones_v, cnt.at[idx_v.at[b]],
                                csem.at[b]).wait()

      issue_load(0, 0)
      issue_load(1, 1)

      def body(g, carry):
        j0 = g * 2
        wait_load(j0, 0)
        start_scat(j0, 0)
        wait_load(j0 + 1, 1)
        start_scat(j0 + 1, 1)
        wait_scat(j0, 0)
        issue_load(j0 + 2, 0)
        wait_scat(j0 + 1, 1)
        issue_load(j0 + 3, 1)
        return carry

      lax.fori_loop(0, -(-per_w // 2), body, 0)

    def dump(sum_out, cnt_out):
      descs = [None, None]
      for k in range(nz):
        b = k % 2
        r = pl.ds(row0 + k * ZROWS, ZROWS)
        if descs[b] is not None:
          descs[b].wait()
        pltpu.sync_copy(acc.at[r, :], rows_v.at[b])
        descs[b] = pltpu.async_copy(rows_v.at[b], sum_out.at[c, r, :],
                                    wsem.at[b])
        pltpu.sync_copy(cnt.at[r, :], cstage)
        pltpu.sync_copy(cstage, cnt_out.at[c, r, :])
      for dsc in descs:
        if dsc is not None:
          dsc.wait()

    zero_acc()
    plsc.subcore_barrier()
    scatter_phase(eattr, ercv, per_w_e, nch_e)
    plsc.subcore_barrier()
    dump(msum, mcnt)
    zero_acc()
    plsc.subcore_barrier()
    scatter_phase(cattr, crcv_h, per_w_c, nch_c)
    plsc.subcore_barrier()
    dump(csum, ccnt)

  return seg_kernel(edge_attr, rcv, cont_attr, crcv,
                    zeros_big, zeros_cnt, ones_cnt)


def _mlp(node_attr, msum, mcnt, csum, ccnt, W1, b1, W2, b2, block_n=2000):
  n, d = node_attr.shape

  def mlp_kernel(x_ref, ms_ref, mc_ref, cs_ref, cc_ref,
                 w1_ref, b1_ref, w2_ref, b2_ref, o_ref):
    ms = ms_ref[0] + ms_ref[1]
    mc = mc_ref[0, :, 0:1] + mc_ref[1, :, 0:1]
    cs = cs_ref[0] + cs_ref[1]
    cc = cc_ref[0, :, 0:1] + cc_ref[1, :, 0:1]
    aggm = ms / jnp.maximum(mc, 1.0)
    aggc = cs / jnp.maximum(cc, 1.0)
    x = x_ref[...]
    w1 = w1_ref[...]
    h = (jnp.dot(x, w1[0:d], preferred_element_type=jnp.float32)
         + jnp.dot(aggm, w1[d:2 * d], preferred_element_type=jnp.float32)
         + jnp.dot(aggc, w1[2 * d:3 * d], preferred_element_type=jnp.float32)
         + b1_ref[...])
    h = jnp.maximum(h, 0.0)
    o_ref[...] = (jnp.dot(h, w2_ref[...], preferred_element_type=jnp.float32)
                  + b2_ref[...])

  return pl.pallas_call(
      mlp_kernel,
      grid=(n // block_n,),
      in_specs=[
          pl.BlockSpec((block_n, d), lambda i: (i, 0)),
          pl.BlockSpec((NC, block_n, d), lambda i: (0, i, 0)),
          pl.BlockSpec((NC, block_n, 16), lambda i: (0, i, 0)),
          pl.BlockSpec((NC, block_n, d), lambda i: (0, i, 0)),
          pl.BlockSpec((NC, block_n, 16), lambda i: (0, i, 0)),
          pl.BlockSpec((3 * d, d), lambda i: (0, 0)),
          pl.BlockSpec((1, d), lambda i: (0, 0)),
          pl.BlockSpec((d, d), lambda i: (0, 0)),
          pl.BlockSpec((1, d), lambda i: (0, 0)),
      ],
      out_specs=pl.BlockSpec((block_n, d), lambda i: (i, 0)),
      out_shape=jax.ShapeDtypeStruct((n, d), jnp.float32),
  )(node_attr, msum, mcnt, csum, ccnt,
    W1, b1.reshape(1, d), W2, b2.reshape(1, d))


def kernel(node_attr, edge_attr, edge_index, edge_contact_attr,
           edge_contact_index, W1, b1, W2, b2):
  num_nodes = node_attr.shape[0]
  rcv = edge_index[1]
  crcv = edge_contact_index[1]
  msum, mcnt, csum, ccnt = _sc_segment_sums(
      edge_attr, rcv, edge_contact_attr, crcv, num_nodes)
  return _mlp(node_attr, msum, mcnt, csum, ccnt, W1, b1, W2, b2)


# speedup vs baseline: 7.6471x; 1.1474x over previous
import functools

import jax
import jax.numpy as jnp
from jax import lax
from jax.experimental import pallas as pl
from jax.experimental.pallas import tpu as pltpu
from jax.experimental.pallas import tpu_sc as plsc

NC = 2
NS = 16
NW = NC * NS
CHUNK = 64
NBUF = 4


def _sc_segment_sums(edge_attr, edge_index, cont_attr, cont_index, num_nodes):
  d = edge_attr.shape[1]
  e = edge_attr.shape[0]
  ec = cont_attr.shape[0]
  nch_e = e // CHUNK
  nch_c = ec // CHUNK
  per_w_e = -(-nch_e // NW)
  per_w_c = -(-nch_c // NW)
  rows_per_tile = -(-num_nodes // (NS * CHUNK)) * CHUNK
  num_nodes = rows_per_tile * NS

  zeros_big = jnp.zeros((CHUNK, d), jnp.float32)
  zeros_cnt = jnp.zeros((CHUNK, 16), jnp.float32)
  ones_cnt = jnp.ones((CHUNK, 16), jnp.float32)

  mesh = plsc.VectorSubcoreMesh(core_axis_name="c", subcore_axis_name="s")

  @functools.partial(
      pl.kernel,
      out_type=(
          jax.ShapeDtypeStruct((NC, num_nodes, d), jnp.float32),
          jax.ShapeDtypeStruct((NC, num_nodes, 16), jnp.float32),
          jax.ShapeDtypeStruct((NC, num_nodes, d), jnp.float32),
          jax.ShapeDtypeStruct((NC, num_nodes, 16), jnp.float32),
      ),
      mesh=mesh,
      compiler_params=pltpu.CompilerParams(use_tc_tiling_on_sc=False),
      scratch_types=[
          pltpu.VMEM_SHARED((num_nodes, d), jnp.float32),
          pltpu.VMEM_SHARED((num_nodes, 16), jnp.float32),
          pltpu.VMEM((NBUF, CHUNK), jnp.int32),
          pltpu.VMEM((NBUF, CHUNK, d), jnp.float32),
          pltpu.VMEM((CHUNK, 16), jnp.float32),
          pltpu.VMEM((CHUNK, 16), jnp.float32),
          pltpu.SemaphoreType.DMA((NBUF,)),
          pltpu.SemaphoreType.DMA((NBUF,)),
          pltpu.SemaphoreType.DMA((NBUF,)),
          pltpu.SemaphoreType.DMA((NBUF,)),
          pltpu.SemaphoreType.DMA((NBUF,)),
      ],
  )
  def seg_kernel(eattr, eidx, cattr, cidx, zb_hbm, zc_hbm, ones_hbm,
                 msum, mcnt, csum, ccnt,
                 acc, cnt, idx_v, rows_v, ones_v, cstage,
                 isem, rsem, ssem, csem, wsem):
    c = lax.axis_index("c")
    s = lax.axis_index("s")
    w = c * NS + s
    row0 = s * rows_per_tile
    nz = rows_per_tile // CHUNK

    pltpu.sync_copy(ones_hbm, ones_v)

    def zero_acc():
      pltpu.sync_copy(zb_hbm, rows_v.at[0])
      pltpu.sync_copy(zc_hbm, cstage)
      descs = []
      for k in range(nz):
        r = pl.ds(row0 + k * CHUNK, CHUNK)
        descs.append(pltpu.async_copy(rows_v.at[0], acc.at[r, :],
                                      wsem.at[0]))
        descs.append(pltpu.async_copy(cstage, cnt.at[r, :], wsem.at[1]))
      for dsc in descs:
        dsc.wait()

    def scatter_phase(attr_hbm, idx_hbm, per_w, nch):
      def issue_load(j, b):
        cid = j * NW + w

        @pl.when(cid < nch)
        def _():
          base = cid * CHUNK
          pltpu.async_copy(idx_hbm.at[1, pl.ds(base, CHUNK)], idx_v.at[b],
                           isem.at[b])
          pltpu.async_copy(attr_hbm.at[pl.ds(base, CHUNK), :], rows_v.at[b],
                           rsem.at[b])

      def wait_load(j, b):
        cid = j * NW + w

        @pl.when(cid < nch)
        def _():
          base = cid * CHUNK
          pltpu.make_async_copy(idx_hbm.at[1, pl.ds(base, CHUNK)],
                                idx_v.at[b], isem.at[b]).wait()
          pltpu.make_async_copy(attr_hbm.at[pl.ds(base, CHUNK), :],
                                rows_v.at[b], rsem.at[b]).wait()

      def start_scat(j, b):
        cid = j * NW + w

        @pl.when(cid < nch)
        def _():
          pltpu.async_copy(rows_v.at[b], acc.at[idx_v.at[b]], ssem.at[b],
                           add=True)
          pltpu.async_copy(ones_v, cnt.at[idx_v.at[b]], csem.at[b],
                           add=True)

      def wait_scat(j, b):
        cid = j * NW + w

        @pl.when(cid < nch)
        def _():
          pltpu.make_async_copy(rows_v.at[b], acc.at[idx_v.at[b]],
                                ssem.at[b]).wait()
          pltpu.make_async_copy(ones_v, cnt.at[idx_v.at[b]],
                                csem.at[b]).wait()

      for b in range(NBUF):
        issue_load(b, b)

      def body(g, carry):
        j0 = g * NBUF
        for b in range(NBUF):
          wait_load(j0 + b, b)
          start_scat(j0 + b, b)
        for b in range(NBUF):
          wait_scat(j0 + b, b)
          issue_load(j0 + NBUF + b, b)
        return carry

      lax.fori_loop(0, -(-per_w // NBUF), body, 0)

    def dump(sum_out, cnt_out):
      descs = [None] * NBUF
      for k in range(nz):
        b = k % NBUF
        r = pl.ds(row0 + k * CHUNK, CHUNK)
        if descs[b] is not None:
          descs[b].wait()
        pltpu.sync_copy(acc.at[r, :], rows_v.at[b])
        descs[b] = pltpu.async_copy(rows_v.at[b], sum_out.at[c, r, :],
                                    wsem.at[b])
        pltpu.sync_copy(cnt.at[r, :], cstage)
        pltpu.sync_copy(cstage, cnt_out.at[c, r, :])
      for dsc in descs:
        if dsc is not None:
          dsc.wait()

    zero_acc()
    plsc.subcore_barrier()
    scatter_phase(eattr, eidx, per_w_e, nch_e)
    plsc.subcore_barrier()
    dump(msum, mcnt)
    plsc.subcore_barrier()
    scatter_phase(cattr, cidx, per_w_c, nch_c)
    plsc.subcore_barrier()
    dump(csum, ccnt)

  return seg_kernel(edge_attr, edge_index, cont_attr, cont_index,
                    zeros_big, zeros_cnt, ones_cnt)


def _mlp(node_attr, msum, mcnt, csum, ccnt, W1, b1, W2, b2, block_n=2000):
  n, d = node_attr.shape

  def mlp_kernel(x_ref, ms_ref, mc_ref, cs_ref, cc_ref,
                 w1_ref, b1_ref, w2_ref, b2_ref, o_ref):
    ms = ms_ref[0] + ms_ref[1]
    mc = mc_ref[0, :, 0:1] + mc_ref[1, :, 0:1]
    cs = cs_ref[0] + cs_ref[1] - ms
    cc = cc_ref[0, :, 0:1] + cc_ref[1, :, 0:1] - mc
    aggm = ms / jnp.maximum(mc, 1.0)
    aggc = cs / jnp.maximum(cc, 1.0)
    x = x_ref[...]
    w1 = w1_ref[...]
    h = (jnp.dot(x, w1[0:d], preferred_element_type=jnp.float32)
         + jnp.dot(aggm, w1[d:2 * d], preferred_element_type=jnp.float32)
         + jnp.dot(aggc, w1[2 * d:3 * d], preferred_element_type=jnp.float32)
         + b1_ref[...])
    h = jnp.maximum(h, 0.0)
    o_ref[...] = (jnp.dot(h, w2_ref[...], preferred_element_type=jnp.float32)
                  + b2_ref[...])

  return pl.pallas_call(
      mlp_kernel,
      grid=(n // block_n,),
      in_specs=[
          pl.BlockSpec((block_n, d), lambda i: (i, 0)),
          pl.BlockSpec((NC, block_n, d), lambda i: (0, i, 0)),
          pl.BlockSpec((NC, block_n, 16), lambda i: (0, i, 0)),
          pl.BlockSpec((NC, block_n, d), lambda i: (0, i, 0)),
          pl.BlockSpec((NC, block_n, 16), lambda i: (0, i, 0)),
          pl.BlockSpec((3 * d, d), lambda i: (0, 0)),
          pl.BlockSpec((1, d), lambda i: (0, 0)),
          pl.BlockSpec((d, d), lambda i: (0, 0)),
          pl.BlockSpec((1, d), lambda i: (0, 0)),
      ],
      out_specs=pl.BlockSpec((block_n, d), lambda i: (i, 0)),
      out_shape=jax.ShapeDtypeStruct((n, d), jnp.float32),
  )(node_attr, msum, mcnt, csum, ccnt,
    W1, b1.reshape(1, d), W2, b2.reshape(1, d))


def kernel(node_attr, edge_attr, edge_index, edge_contact_attr,
           edge_contact_index, W1, b1, W2, b2):
  num_nodes = node_attr.shape[0]
  msum, mcnt, csum, ccnt = _sc_segment_sums(
      edge_attr, edge_index, edge_contact_attr, edge_contact_index, num_nodes)
  return _mlp(node_attr, msum, mcnt, csum, ccnt, W1, b1, W2, b2)


# speedup vs baseline: 8.3561x; 1.0927x over previous
import functools

import jax
import jax.numpy as jnp
from jax import lax
from jax.experimental import pallas as pl
from jax.experimental.pallas import tpu as pltpu
from jax.experimental.pallas import tpu_sc as plsc

NC = 2
NS = 16
NW = NC * NS
CHUNK = 64
NBUF = 4


def _sc_segment_sums(edge_attr, edge_index, cont_attr, cont_index, num_nodes):
  d = edge_attr.shape[1]
  e = edge_attr.shape[0]
  ec = cont_attr.shape[0]
  nch_e = e // CHUNK
  nch_c = ec // CHUNK
  per_w_e = -(-nch_e // NW)
  per_w_c = -(-nch_c // NW)
  rows_per_tile = -(-num_nodes // (NS * CHUNK)) * CHUNK
  num_nodes = rows_per_tile * NS

  zeros_big = jnp.zeros((CHUNK, d), jnp.float32)
  zeros_hist = jnp.zeros((num_nodes,), jnp.float32)

  mesh = plsc.VectorSubcoreMesh(core_axis_name="c", subcore_axis_name="s")

  @functools.partial(
      pl.kernel,
      out_type=(
          jax.ShapeDtypeStruct((NC, num_nodes, d), jnp.float32),
          jax.ShapeDtypeStruct((NC, NS, num_nodes), jnp.float32),
          jax.ShapeDtypeStruct((NC, num_nodes, d), jnp.float32),
          jax.ShapeDtypeStruct((NC, NS, num_nodes), jnp.float32),
      ),
      mesh=mesh,
      compiler_params=pltpu.CompilerParams(use_tc_tiling_on_sc=False,
                                           needs_layout_passes=False),
      scratch_types=[
          pltpu.VMEM_SHARED((num_nodes, d), jnp.float32),
          pltpu.VMEM((NBUF, CHUNK), jnp.int32),
          pltpu.VMEM((NBUF, CHUNK, d), jnp.float32),
          pltpu.VMEM((num_nodes,), jnp.float32),
          pltpu.SemaphoreType.DMA((NBUF,)),
          pltpu.SemaphoreType.DMA((NBUF,)),
          pltpu.SemaphoreType.DMA((NBUF,)),
          pltpu.SemaphoreType.DMA((NBUF,)),
      ],
  )
  def seg_kernel(eattr, eidx, cattr, cidx, zb_hbm, zh_hbm,
                 msum, mcnt, csum, ccnt,
                 acc, idx_v, rows_v, hist,
                 isem, rsem, ssem, wsem):
    c = lax.axis_index("c")
    s = lax.axis_index("s")
    w = c * NS + s
    row0 = s * rows_per_tile
    nz = rows_per_tile // CHUNK

    pltpu.sync_copy(zh_hbm, hist)

    def zero_acc():
      pltpu.sync_copy(zb_hbm, rows_v.at[0])
      descs = []
      for k in range(nz):
        r = pl.ds(row0 + k * CHUNK, CHUNK)
        descs.append(pltpu.async_copy(rows_v.at[0], acc.at[r, :],
                                      wsem.at[0]))
      for dsc in descs:
        dsc.wait()

    def scatter_phase(attr_hbm, idx_hbm, per_w, nch):
      def issue_load(j, b):
        cid = j * NW + w

        @pl.when(cid < nch)
        def _():
          base = cid * CHUNK
          pltpu.async_copy(idx_hbm.at[1, pl.ds(base, CHUNK)], idx_v.at[b],
                           isem.at[b])
          pltpu.async_copy(attr_hbm.at[pl.ds(base, CHUNK), :], rows_v.at[b],
                           rsem.at[b])

      def wait_load(j, b):
        cid = j * NW + w

        @pl.when(cid < nch)
        def _():
          base = cid * CHUNK
          pltpu.make_async_copy(idx_hbm.at[1, pl.ds(base, CHUNK)],
                                idx_v.at[b], isem.at[b]).wait()
          pltpu.make_async_copy(attr_hbm.at[pl.ds(base, CHUNK), :],
                                rows_v.at[b], rsem.at[b]).wait()

      def start_scat(j, b):
        cid = j * NW + w

        @pl.when(cid < nch)
        def _():
          pltpu.async_copy(rows_v.at[b], acc.at[idx_v.at[b]], ssem.at[b],
                           add=True)

      def hist_update(j, b):
        cid = j * NW + w

        @pl.when(cid < nch)
        def _():
          for v in range(CHUNK // 16):
            iv = idx_v[b, pl.ds(v * 16, 16)]
            cnts, lmask = plsc.scan_count(iv)
            plsc.addupdate_scatter(hist, [iv], cnts.astype(jnp.float32),
                                   mask=lmask)

      def wait_scat(j, b):
        cid = j * NW + w

        @pl.when(cid < nch)
        def _():
          pltpu.make_async_copy(rows_v.at[b], acc.at[idx_v.at[b]],
                                ssem.at[b]).wait()

      for b in range(NBUF):
        issue_load(b, b)

      def body(g, carry):
        j0 = g * NBUF
        for b in range(NBUF):
          wait_load(j0 + b, b)
          start_scat(j0 + b, b)
          hist_update(j0 + b, b)
        for b in range(NBUF):
          wait_scat(j0 + b, b)
          issue_load(j0 + NBUF + b, b)
        return carry

      lax.fori_loop(0, -(-per_w // NBUF), body, 0)

    def dump(sum_out, cnt_out):
      descs = [None] * NBUF
      for k in range(nz):
        b = k % NBUF
        r = pl.ds(row0 + k * CHUNK, CHUNK)
        if descs[b] is not None:
          descs[b].wait()
        pltpu.sync_copy(acc.at[r, :], rows_v.at[b])
        descs[b] = pltpu.async_copy(rows_v.at[b], sum_out.at[c, r, :],
                                    wsem.at[b])
      for dsc in descs:
        if dsc is not None:
          dsc.wait()
      pltpu.sync_copy(hist, cnt_out.at[c, s, :])

    zero_acc()
    plsc.subcore_barrier()
    scatter_phase(eattr, eidx, per_w_e, nch_e)
    plsc.subcore_barrier()
    dump(msum, mcnt)
    plsc.subcore_barrier()
    scatter_phase(cattr, cidx, per_w_c, nch_c)
    plsc.subcore_barrier()
    dump(csum, ccnt)

  return seg_kernel(edge_attr, edge_index, cont_attr, cont_index,
                    zeros_big, zeros_hist)


def _mlp(node_attr, msum, mcnt, csum, ccnt, W1, b1, W2, b2, block_n=2048):
  n, d = node_attr.shape

  def mlp_kernel(x_ref, ms_ref, mc_ref, cs_ref, cc_ref,
                 w1_ref, b1_ref, w2_ref, b2_ref, o_ref):
    ms = ms_ref[0] + ms_ref[1]
    ones_w = jnp.ones((NC * NS, 1), jnp.float32)
    mc2 = mc_ref[...].reshape(NC * NS, -1)
    cc2 = cc_ref[...].reshape(NC * NS, -1)
    mc = lax.dot_general(mc2, ones_w, (((0,), (0,)), ((), ())),
                         preferred_element_type=jnp.float32)
    cs = cs_ref[0] + cs_ref[1] - ms
    cc = lax.dot_general(cc2, ones_w, (((0,), (0,)), ((), ())),
                         preferred_element_type=jnp.float32) - mc
    aggm = ms / jnp.maximum(mc, 1.0)
    aggc = cs / jnp.maximum(cc, 1.0)
    x = x_ref[...]
    w1 = w1_ref[...]
    h = (jnp.dot(x, w1[0:d], preferred_element_type=jnp.float32)
         + jnp.dot(aggm, w1[d:2 * d], preferred_element_type=jnp.float32)
         + jnp.dot(aggc, w1[2 * d:3 * d], preferred_element_type=jnp.float32)
         + b1_ref[...])
    h = jnp.maximum(h, 0.0)
    o_ref[...] = (jnp.dot(h, w2_ref[...], preferred_element_type=jnp.float32)
                  + b2_ref[...])

  return pl.pallas_call(
      mlp_kernel,
      grid=(-(-n // block_n),),
      in_specs=[
          pl.BlockSpec((block_n, d), lambda i: (i, 0)),
          pl.BlockSpec((NC, block_n, d), lambda i: (0, i, 0)),
          pl.BlockSpec((NC, NS, block_n), lambda i: (0, 0, i)),
          pl.BlockSpec((NC, block_n, d), lambda i: (0, i, 0)),
          pl.BlockSpec((NC, NS, block_n), lambda i: (0, 0, i)),
          pl.BlockSpec((3 * d, d), lambda i: (0, 0)),
          pl.BlockSpec((1, d), lambda i: (0, 0)),
          pl.BlockSpec((d, d), lambda i: (0, 0)),
          pl.BlockSpec((1, d), lambda i: (0, 0)),
      ],
      out_specs=pl.BlockSpec((block_n, d), lambda i: (i, 0)),
      out_shape=jax.ShapeDtypeStruct((n, d), jnp.float32),
  )(node_attr, msum, mcnt, csum, ccnt,
    W1, b1.reshape(1, d), W2, b2.reshape(1, d))


def kernel(node_attr, edge_attr, edge_index, edge_contact_attr,
           edge_contact_index, W1, b1, W2, b2):
  num_nodes = node_attr.shape[0]
  msum, mcnt, csum, ccnt = _sc_segment_sums(
      edge_attr, edge_index, edge_contact_attr, edge_contact_index, num_nodes)
  return _mlp(node_attr, msum, mcnt, csum, ccnt, W1, b1, W2, b2)
